# Initial kernel scaffold; baseline (speedup 1.0000x reference)
import jax
import jax.numpy as jnp
from jax.experimental import pallas as pl


def kernel(points1, points2):
    raise NotImplementedError("write your pallas kernel here")


# speedup vs baseline: 2.5648x; 2.5648x over previous
import functools

import jax
import jax.numpy as jnp
from jax.experimental import pallas as pl
from jax.experimental.pallas import tpu as pltpu


def _chamfer_body(xp_ref, ytp_ref, out_ref, colmin_ref, rowacc_ref):
    i = pl.program_id(0)
    xb = xp_ref[...]
    yt = ytp_ref[...]
    s = jax.lax.dot_general(
        xb.astype(jnp.bfloat16), yt.astype(jnp.bfloat16),
        (((1,), (0,)), ((), ())),
        preferred_element_type=jnp.float32)
    x2 = jnp.sum(xb * xb, axis=1, keepdims=True)
    y2 = jnp.sum(yt * yt, axis=0, keepdims=True)
    d2 = jnp.maximum(x2 + y2 - 2.0 * s, 0.0)
    rowmin_sum = jnp.sum(jnp.min(d2, axis=1))
    partial_col = jnp.min(d2, axis=0, keepdims=True)

    @pl.when(i == 0)
    def _init():
        colmin_ref[...] = partial_col
        rowacc_ref[0] = rowmin_sum

    @pl.when(i > 0)
    def _accum():
        colmin_ref[...] = jnp.minimum(colmin_ref[...], partial_col)
        rowacc_ref[0] = rowacc_ref[0] + rowmin_sum

    @pl.when(i == pl.num_programs(0) - 1)
    def _final():
        total = rowacc_ref[0] + jnp.sum(colmin_ref[...])
        out_ref[...] = jnp.full((1, 1), total, dtype=jnp.float32)


def kernel(points1, points2):
    _, n, d = points1.shape
    _, m, _ = points2.shape
    p1 = points1.reshape(n, d)
    p2 = points2.reshape(m, d)
    xp = jnp.pad(p1, ((0, 0), (0, 8 - d)))
    ytp = jnp.pad(p2, ((0, 0), (0, 8 - d))).T
    r = 512
    grid = n // r
    out = pl.pallas_call(
        _chamfer_body,
        grid=(grid,),
        in_specs=[
            pl.BlockSpec((r, 8), lambda i: (i, 0)),
            pl.BlockSpec((8, m), lambda i: (0, 0)),
        ],
        out_specs=pl.BlockSpec((1, 1), lambda i: (0, 0)),
        out_shape=jax.ShapeDtypeStruct((1, 1), jnp.float32),
        scratch_shapes=[
            pltpu.VMEM((1, m), jnp.float32),
            pltpu.SMEM((1,), jnp.float32),
        ],
    )(xp, ytp)
    return out[0, 0]
